# baseline (speedup 1.0000x reference)
import jax
import jax.numpy as jnp
from jax.experimental import pallas as pl
from jax.experimental.pallas import tpu as pltpu

IN_F = 1024
OUT_F = 1024
KSEL = 16
BATCH = 256
CHUNKS = IN_F // KSEL


def _oddeven_pairs(n):
    pairs = []

    def merge(lo, m, r):
        step = r * 2
        if step < m:
            merge(lo, m, step)
            merge(lo + r, m, step)
            for i in range(lo + r, lo + m - r, step):
                pairs.append((i, i + r))
        else:
            pairs.append((lo, lo + r))

    def sort(lo, m):
        if m > 1:
            half = m // 2
            sort(lo, half)
            sort(lo + half, half)
            merge(lo, m, 1)

    sort(0, n)
    return pairs


_SORT16 = _oddeven_pairs(KSEL)


def _sort_desc(planes):
    planes = list(planes)
    for i, j in _SORT16:
        a, b = planes[i], planes[j]
        planes[i] = jnp.maximum(a, b)
        planes[j] = jnp.minimum(a, b)
    return planes


def _bitonic_merge(planes, descending):
    planes = list(planes)
    for d in (8, 4, 2, 1):
        for i in range(KSEL):
            if i & d == 0:
                a, b = planes[i], planes[i + d]
                if descending:
                    planes[i] = jnp.maximum(a, b)
                    planes[i + d] = jnp.minimum(a, b)
                else:
                    planes[i] = jnp.minimum(a, b)
                    planes[i + d] = jnp.maximum(a, b)
    return planes


def _merge_top(a, b):
    c = [jnp.maximum(a[p], b[KSEL - 1 - p]) for p in range(KSEL)]
    return _bitonic_merge(c, descending=True)


def _merge_bot(a, b):
    c = [jnp.minimum(a[p], b[KSEL - 1 - p]) for p in range(KSEL)]
    return _bitonic_merge(c, descending=False)


def _tree_sum(planes):
    vals = list(planes)
    while len(vals) > 1:
        nxt = [vals[i] + vals[i + 1] for i in range(0, len(vals) - 1, 2)]
        if len(vals) % 2:
            nxt.append(vals[-1])
        vals = nxt
    return vals[0]


_LEVEL_NODES = [CHUNKS >> (l + 1) for l in range(6)]
_LEVEL_OFF = [0]
for _n in _LEVEL_NODES[:-1]:
    _LEVEL_OFF.append(_LEVEL_OFF[-1] + _n * KSEL)


def _body(x_ref, wt_ref, out_ref, chunk_s, top_s, bot_s):
    def sort_one(c):
        base = c * KSEL
        w16 = wt_ref[pl.ds(base, KSEL), :, :]
        planes = [x_ref[0, 0, base + p] * w16[p] for p in range(KSEL)]
        ssum = _tree_sum(planes)
        sp = _sort_desc(planes)
        chunk_s[pl.ds(base, KSEL), :, :] = jnp.stack(sp)
        return ssum

    def phase_a(i, tot):
        return tot + sort_one(2 * i) + sort_one(2 * i + 1)

    tot = jax.lax.fori_loop(0, CHUNKS // 2, phase_a,
                            jnp.zeros((8, 128), jnp.float32))

    def load16(ref, off):
        v = ref[pl.ds(off, KSEL), :, :]
        return [v[p] for p in range(KSEL)]

    def level0(j, tot):
        a = load16(chunk_s, 32 * j)
        b = load16(chunk_s, 32 * j + KSEL)
        top_s[pl.ds(KSEL * j, KSEL), :, :] = jnp.stack(_merge_top(a, b))
        ar = a[::-1]
        br = b[::-1]
        bot_s[pl.ds(KSEL * j, KSEL), :, :] = jnp.stack(_merge_bot(ar, br))
        return tot

    tot = jax.lax.fori_loop(0, _LEVEL_NODES[0], level0, tot)

    for lvl in range(1, 5):
        src = _LEVEL_OFF[lvl - 1]
        dst = _LEVEL_OFF[lvl]

        def level_l(j, tot, src=src, dst=dst):
            a = load16(top_s, src + 32 * j)
            b = load16(top_s, src + 32 * j + KSEL)
            top_s[pl.ds(dst + KSEL * j, KSEL), :, :] = \
                jnp.stack(_merge_top(a, b))
            a2 = load16(bot_s, src + 32 * j)
            b2 = load16(bot_s, src + 32 * j + KSEL)
            bot_s[pl.ds(dst + KSEL * j, KSEL), :, :] = \
                jnp.stack(_merge_bot(a2, b2))
            return tot

        tot = jax.lax.fori_loop(0, _LEVEL_NODES[lvl], level_l, tot)

    src = _LEVEL_OFF[4]
    ta = load16(top_s, src)
    tb = load16(top_s, src + KSEL)
    tsum = _tree_sum(_merge_top(ta, tb))
    ba = load16(bot_s, src)
    bb = load16(bot_s, src + KSEL)
    bsum = _tree_sum(_merge_bot(ba, bb))

    out_ref[0] = tot - tsum + bsum


@jax.jit
def kernel(x, W):
    wt = W.T.reshape(IN_F, 8, 128)
    x3 = x.reshape(BATCH, 1, IN_F)
    n_tree = _LEVEL_OFF[-1] + KSEL
    out3 = pl.pallas_call(
        _body,
        grid=(BATCH,),
        in_specs=[
            pl.BlockSpec((1, 1, IN_F), lambda b: (b, 0, 0),
                         memory_space=pltpu.SMEM),
            pl.BlockSpec((IN_F, 8, 128), lambda b: (0, 0, 0)),
        ],
        out_specs=pl.BlockSpec((1, 8, 128), lambda b: (b, 0, 0)),
        out_shape=jax.ShapeDtypeStruct((BATCH, 8, 128), jnp.float32),
        scratch_shapes=[
            pltpu.VMEM((IN_F, 8, 128), jnp.float32),
            pltpu.VMEM((n_tree, 8, 128), jnp.float32),
            pltpu.VMEM((n_tree, 8, 128), jnp.float32),
        ],
    )(x3, wt)
    return out3.reshape(BATCH, OUT_F)


# speedup vs baseline: 1.0867x; 1.0867x over previous
import jax
import jax.numpy as jnp
from jax.experimental import pallas as pl
from jax.experimental.pallas import tpu as pltpu

IN_F = 1024
OUT_F = 1024
KSEL = 16
BATCH = 256
CHUNKS = IN_F // KSEL


def _oddeven_pairs(n):
    sort_pairs = []
    merge_pairs = []

    def merge(lo, m, r, out):
        step = r * 2
        if step < m:
            merge(lo, m, step, out)
            merge(lo + r, m, step, out)
            for i in range(lo + r, lo + m - r, step):
                out.append((i, i + r))
        else:
            out.append((lo, lo + r))

    def sort(lo, m):
        if m > 1:
            half = m // 2
            sort(lo, half)
            sort(lo + half, half)
            merge(lo, m, 1, sort_pairs)

    sort(0, n)
    merge(0, 2 * n, 1, merge_pairs)
    return sort_pairs, merge_pairs


_SORT16, _MERGE32 = _oddeven_pairs(KSEL)


def _ce_desc(planes, pairs):
    planes = list(planes)
    for i, j in pairs:
        a, b = planes[i], planes[j]
        planes[i] = jnp.maximum(a, b)
        planes[j] = jnp.minimum(a, b)
    return planes


def _bitonic_merge(planes, descending):
    planes = list(planes)
    for d in (8, 4, 2, 1):
        for i in range(KSEL):
            if i & d == 0:
                a, b = planes[i], planes[i + d]
                if descending:
                    planes[i] = jnp.maximum(a, b)
                    planes[i + d] = jnp.minimum(a, b)
                else:
                    planes[i] = jnp.minimum(a, b)
                    planes[i + d] = jnp.maximum(a, b)
    return planes


def _tree_sum(planes):
    vals = list(planes)
    while len(vals) > 1:
        nxt = [vals[i] + vals[i + 1] for i in range(0, len(vals) - 1, 2)]
        if len(vals) % 2:
            nxt.append(vals[-1])
        vals = nxt
    return vals[0]


def _body(x_ref, xr_ref, wt_ref, w2_ref, out_ref):
    def step(i, carry):
        top, bot = carry
        base = 2 * KSEL * i
        w32 = wt_ref[pl.ds(base, 2 * KSEL), :, :]
        planes = [x_ref[0, 0, base + p] * w32[p] for p in range(2 * KSEL)]
        sa = _ce_desc(planes[:KSEL], _SORT16)
        sb = _ce_desc(planes[KSEL:], _SORT16)
        run = _ce_desc(sa + sb, _MERGE32)
        ctop = [jnp.maximum(top[p], run[KSEL - 1 - p]) for p in range(KSEL)]
        top = tuple(_bitonic_merge(ctop, descending=True))
        cbot = [jnp.minimum(bot[p], run[KSEL + p]) for p in range(KSEL)]
        bot = tuple(_bitonic_merge(cbot, descending=False))
        return top, bot

    neg = jnp.full((8, 128), -jnp.inf, jnp.float32)
    pos = jnp.full((8, 128), jnp.inf, jnp.float32)
    top, bot = jax.lax.fori_loop(0, CHUNKS // 2, step,
                                 ((neg,) * KSEL, (pos,) * KSEL))
    corr = _tree_sum(top) - _tree_sum(bot)
    dot = jax.lax.dot_general(
        xr_ref[0], w2_ref[...], (((1,), (0,)), ((), ())),
        preferred_element_type=jnp.float32)
    out_ref[0] = dot - corr.reshape(1, OUT_F)


@jax.jit
def kernel(x, W):
    wt2 = W.T
    wt3 = wt2.reshape(IN_F, 8, 128)
    x3 = x.reshape(BATCH, 1, IN_F)
    out3 = pl.pallas_call(
        _body,
        grid=(BATCH,),
        in_specs=[
            pl.BlockSpec((1, 1, IN_F), lambda b: (b, 0, 0),
                         memory_space=pltpu.SMEM),
            pl.BlockSpec((1, 1, IN_F), lambda b: (b, 0, 0)),
            pl.BlockSpec((IN_F, 8, 128), lambda b: (0, 0, 0)),
            pl.BlockSpec((IN_F, OUT_F), lambda b: (0, 0)),
        ],
        out_specs=pl.BlockSpec((1, 1, OUT_F), lambda b: (b, 0, 0)),
        out_shape=jax.ShapeDtypeStruct((BATCH, 1, OUT_F), jnp.float32),
    )(x3, x3, wt3, wt2)
    return out3.reshape(BATCH, OUT_F)


# speedup vs baseline: 1.2528x; 1.1528x over previous
import jax
import jax.numpy as jnp
from jax.experimental import pallas as pl
from jax.experimental.pallas import tpu as pltpu

IN_F = 1024
OUT_F = 1024
KSEL = 16
BATCH = 256
CHUNKS = IN_F // KSEL


def _oddeven_pairs(n):
    sort_pairs = []
    merge_pairs = []

    def merge(lo, m, r, out):
        step = r * 2
        if step < m:
            merge(lo, m, step, out)
            merge(lo + r, m, step, out)
            for i in range(lo + r, lo + m - r, step):
                out.append((i, i + r))
        else:
            out.append((lo, lo + r))

    def sort(lo, m):
        if m > 1:
            half = m // 2
            sort(lo, half)
            sort(lo + half, half)
            merge(lo, m, 1, sort_pairs)

    sort(0, n)
    merge(0, 2 * n, 1, merge_pairs)
    return sort_pairs, merge_pairs


_SORT16, _MERGE32 = _oddeven_pairs(KSEL)


def _ce_desc(planes, pairs):
    planes = list(planes)
    for i, j in pairs:
        a, b = planes[i], planes[j]
        planes[i] = jnp.maximum(a, b)
        planes[j] = jnp.minimum(a, b)
    return planes


def _bitonic_merge(planes, descending):
    planes = list(planes)
    for d in (8, 4, 2, 1):
        for i in range(KSEL):
            if i & d == 0:
                a, b = planes[i], planes[i + d]
                if descending:
                    planes[i] = jnp.maximum(a, b)
                    planes[i + d] = jnp.minimum(a, b)
                else:
                    planes[i] = jnp.minimum(a, b)
                    planes[i + d] = jnp.maximum(a, b)
    return planes


def _tree_sum(planes):
    vals = list(planes)
    while len(vals) > 1:
        nxt = [vals[i] + vals[i + 1] for i in range(0, len(vals) - 1, 2)]
        if len(vals) % 2:
            nxt.append(vals[-1])
        vals = nxt
    return vals[0]


def _mm_body(x_ref, w2_ref, out_ref):
    out_ref[...] = jax.lax.dot_general(
        x_ref[...], w2_ref[...], (((1,), (0,)), ((), ())),
        preferred_element_type=jnp.float32)


def _body(x_ref, wt_ref, out_ref):
    def step(i, carry):
        top, bot = carry
        base = 2 * KSEL * i
        w32 = wt_ref[pl.ds(base, 2 * KSEL), :, :]
        planes = [x_ref[0, 0, base + p] * w32[p] for p in range(2 * KSEL)]
        sa = _ce_desc(planes[:KSEL], _SORT16)
        sb = _ce_desc(planes[KSEL:], _SORT16)
        run = _ce_desc(sa + sb, _MERGE32)
        ctop = [jnp.maximum(top[p], run[KSEL - 1 - p]) for p in range(KSEL)]
        top = tuple(_bitonic_merge(ctop, descending=True))
        cbot = [jnp.minimum(bot[p], run[KSEL + p]) for p in range(KSEL)]
        bot = tuple(_bitonic_merge(cbot, descending=False))
        return top, bot

    neg = jnp.full((8, 128), -jnp.inf, jnp.float32)
    pos = jnp.full((8, 128), jnp.inf, jnp.float32)
    top, bot = jax.lax.fori_loop(0, CHUNKS // 2, step,
                                 ((neg,) * KSEL, (pos,) * KSEL))
    out_ref[0] = _tree_sum(top) - _tree_sum(bot)


@jax.jit
def kernel(x, W):
    wt2 = W.T
    wt3 = wt2.reshape(IN_F, 8, 128)
    x3 = x.reshape(BATCH, 1, IN_F)
    mm = pl.pallas_call(
        _mm_body,
        in_specs=[
            pl.BlockSpec((BATCH, IN_F), lambda: (0, 0)),
            pl.BlockSpec((IN_F, OUT_F), lambda: (0, 0)),
        ],
        out_specs=pl.BlockSpec((BATCH, OUT_F), lambda: (0, 0)),
        out_shape=jax.ShapeDtypeStruct((BATCH, OUT_F), jnp.float32),
    )(x, wt2)
    corr3 = pl.pallas_call(
        _body,
        grid=(BATCH,),
        in_specs=[
            pl.BlockSpec((1, 1, IN_F), lambda b: (b, 0, 0),
                         memory_space=pltpu.SMEM),
            pl.BlockSpec((IN_F, 8, 128), lambda b: (0, 0, 0)),
        ],
        out_specs=pl.BlockSpec((1, 8, 128), lambda b: (b, 0, 0)),
        out_shape=jax.ShapeDtypeStruct((BATCH, 8, 128), jnp.float32),
    )(x3, wt3)
    return mm - corr3.reshape(BATCH, OUT_F)


# speedup vs baseline: 1.3267x; 1.0590x over previous
import jax
import jax.numpy as jnp
from jax.experimental import pallas as pl
from jax.experimental.pallas import tpu as pltpu

IN_F = 1024
OUT_F = 1024
KSEL = 16
BATCH = 256
CHUNKS = IN_F // KSEL


def _oddeven_pairs(n):
    sort_pairs = []
    merge_pairs = []

    def merge(lo, m, r, out):
        step = r * 2
        if step < m:
            merge(lo, m, step, out)
            merge(lo + r, m, step, out)
            for i in range(lo + r, lo + m - r, step):
                out.append((i, i + r))
        else:
            out.append((lo, lo + r))

    def sort(lo, m):
        if m > 1:
            half = m // 2
            sort(lo, half)
            sort(lo + half, half)
            merge(lo, m, 1, sort_pairs)

    sort(0, n)
    merge(0, 2 * n, 1, merge_pairs)
    return sort_pairs, merge_pairs


_SORT16, _MERGE32 = _oddeven_pairs(KSEL)


def _ce_desc(planes, pairs):
    planes = list(planes)
    for i, j in pairs:
        a, b = planes[i], planes[j]
        planes[i] = jnp.maximum(a, b)
        planes[j] = jnp.minimum(a, b)
    return planes


def _bitonic_merge(planes, descending):
    planes = list(planes)
    for d in (8, 4, 2, 1):
        for i in range(KSEL):
            if i & d == 0:
                a, b = planes[i], planes[i + d]
                if descending:
                    planes[i] = jnp.maximum(a, b)
                    planes[i + d] = jnp.minimum(a, b)
                else:
                    planes[i] = jnp.minimum(a, b)
                    planes[i + d] = jnp.maximum(a, b)
    return planes


def _tree_sum(planes):
    vals = list(planes)
    while len(vals) > 1:
        nxt = [vals[i] + vals[i + 1] for i in range(0, len(vals) - 1, 2)]
        if len(vals) % 2:
            nxt.append(vals[-1])
        vals = nxt
    return vals[0]


def _mm_body(x_ref, w2_ref, out_ref):
    out_ref[...] = jax.lax.dot_general(
        x_ref[...], w2_ref[...], (((1,), (0,)), ((), ())),
        preferred_element_type=jnp.float32)


def _make_run(x_ref, wt_ref, base):
    w32 = wt_ref[pl.ds(base, 2 * KSEL), :, :]
    planes = [x_ref[0, 0, base + p] * w32[p] for p in range(2 * KSEL)]
    sa = _ce_desc(planes[:KSEL], _SORT16)
    sb = _ce_desc(planes[KSEL:], _SORT16)
    return _ce_desc(sa + sb, _MERGE32)


def _merge_top(a, b):
    c = [jnp.maximum(a[p], b[KSEL - 1 - p]) for p in range(KSEL)]
    return _bitonic_merge(c, descending=True)


def _merge_bot(a, b):
    c = [jnp.minimum(a[p], b[KSEL - 1 - p]) for p in range(KSEL)]
    return _bitonic_merge(c, descending=False)


def _body(x_ref, wt_ref, out_ref):
    def step(i, carry):
        top, bot = carry
        base = 4 * KSEL * i
        r1 = _make_run(x_ref, wt_ref, base)
        r2 = _make_run(x_ref, wt_ref, base + 2 * KSEL)
        ptop = _merge_top(r1[:KSEL], r2[:KSEL])
        pbot = _merge_bot(r1[:KSEL - 1:-1], r2[:KSEL - 1:-1])
        top = tuple(_merge_top(list(top), ptop))
        bot = tuple(_merge_bot(list(bot), pbot))
        return top, bot

    neg = jnp.full((8, 128), -jnp.inf, jnp.float32)
    pos = jnp.full((8, 128), jnp.inf, jnp.float32)
    top, bot = jax.lax.fori_loop(0, CHUNKS // 4, step,
                                 ((neg,) * KSEL, (pos,) * KSEL))
    out_ref[0] = _tree_sum(top) - _tree_sum(bot)


@jax.jit
def kernel(x, W):
    wt2 = W.T
    wt3 = wt2.reshape(IN_F, 8, 128)
    x3 = x.reshape(BATCH, 1, IN_F)
    mm = pl.pallas_call(
        _mm_body,
        in_specs=[
            pl.BlockSpec((BATCH, IN_F), lambda: (0, 0)),
            pl.BlockSpec((IN_F, OUT_F), lambda: (0, 0)),
        ],
        out_specs=pl.BlockSpec((BATCH, OUT_F), lambda: (0, 0)),
        out_shape=jax.ShapeDtypeStruct((BATCH, OUT_F), jnp.float32),
    )(x, wt2)
    corr3 = pl.pallas_call(
        _body,
        grid=(BATCH,),
        in_specs=[
            pl.BlockSpec((1, 1, IN_F), lambda b: (b, 0, 0),
                         memory_space=pltpu.SMEM),
            pl.BlockSpec((IN_F, 8, 128), lambda b: (0, 0, 0)),
        ],
        out_specs=pl.BlockSpec((1, 8, 128), lambda b: (b, 0, 0)),
        out_shape=jax.ShapeDtypeStruct((BATCH, 8, 128), jnp.float32),
    )(x3, wt3)
    return mm - corr3.reshape(BATCH, OUT_F)
